# baseline (speedup 1.0000x reference)
import functools
import math

import jax
import jax.numpy as jnp
from jax import lax
from jax.experimental import pallas as pl
from jax.experimental.pallas import tpu as pltpu
from jax.experimental.pallas import tpu_sc as plsc

D_MODEL = 64
SCALE = math.sqrt(D_MODEL)

_NUM_CORES = 2
_NUM_SUBCORES = 16
_NW = _NUM_CORES * _NUM_SUBCORES

_SUB = 128
_L = 16


@functools.partial(jax.jit, static_argnames=("n",))
def _embed_flat(table, idx, *, n):
    npw = n // _NW
    nsub = npw // _SUB

    mesh = plsc.VectorSubcoreMesh(core_axis_name="c", subcore_axis_name="s")

    @functools.partial(
        pl.kernel,
        out_type=jax.ShapeDtypeStruct((n, D_MODEL), jnp.float32),
        mesh=mesh,
        compiler_params=pltpu.CompilerParams(use_tc_tiling_on_sc=False),
        scratch_types=[
            pltpu.VMEM((npw,), jnp.int32),
            pltpu.VMEM((_SUB, D_MODEL), jnp.float32),
            pltpu.SemaphoreType.DMA,
        ],
    )
    def emb(table_hbm, idx_hbm, out_hbm, idx_v, rows_v, sem):
        worker = lax.axis_index("s") * _NUM_CORES + lax.axis_index("c")
        base = worker * npw
        pltpu.sync_copy(idx_hbm.at[pl.ds(base, npw)], idx_v)

        @pl.loop(0, nsub)
        def _chunk(g):
            pltpu.async_copy(
                table_hbm.at[idx_v.at[pl.ds(g * _SUB, _SUB)]], rows_v, sem
            ).wait()

            @plsc.parallel_loop(0, _SUB)
            def _scale(r):
                for j in range(D_MODEL // _L):
                    sl = pl.ds(j * _L, _L)
                    rows_v[r, sl] = rows_v[r, sl] * SCALE

            pltpu.sync_copy(
                rows_v, out_hbm.at[pl.ds(base + g * _SUB, _SUB)]
            )

    return emb(table, idx)


def kernel(x, table):
    n = x.size
    idx = x.reshape(n).astype(jnp.int32)
    out = _embed_flat(table, idx, n=n)
    return out.reshape(x.shape + (D_MODEL,))


# speedup vs baseline: 1.0643x; 1.0643x over previous
import functools
import math

import jax
import jax.numpy as jnp
from jax import lax
from jax.experimental import pallas as pl
from jax.experimental.pallas import tpu as pltpu
from jax.experimental.pallas import tpu_sc as plsc

D_MODEL = 64
SCALE = math.sqrt(D_MODEL)

_NUM_CORES = 2
_NUM_SUBCORES = 16
_NW = _NUM_CORES * _NUM_SUBCORES

_SUB = 128
_L = 16
_NBUF = 2


@functools.partial(jax.jit, static_argnames=("n",))
def _embed_flat(table, idx, *, n):
    npw = n // _NW
    nsub = npw // _SUB

    mesh = plsc.VectorSubcoreMesh(core_axis_name="c", subcore_axis_name="s")

    @functools.partial(
        pl.kernel,
        out_type=jax.ShapeDtypeStruct((n, D_MODEL), jnp.float32),
        mesh=mesh,
        compiler_params=pltpu.CompilerParams(use_tc_tiling_on_sc=False),
        scratch_types=[
            pltpu.VMEM((npw,), jnp.int32),
            pltpu.VMEM((_SUB, D_MODEL), jnp.float32),
            pltpu.VMEM((_SUB, D_MODEL), jnp.float32),
            pltpu.VMEM((_SUB, D_MODEL), jnp.float32),
            pltpu.VMEM((_SUB, D_MODEL), jnp.float32),
            pltpu.SemaphoreType.DMA,
            pltpu.SemaphoreType.DMA,
            pltpu.SemaphoreType.DMA,
            pltpu.SemaphoreType.DMA,
        ],
    )
    def emb(table_hbm, idx_hbm, out_hbm, idx_v,
            gb0, gb1, ob0, ob1, gs0, gs1, ss0, ss1):
        worker = lax.axis_index("s") * _NUM_CORES + lax.axis_index("c")
        base = worker * npw
        pltpu.sync_copy(idx_hbm.at[pl.ds(base, npw)], idx_v)

        gbuf = (gb0, gb1)
        obuf = (ob0, ob1)
        gsem = (gs0, gs1)
        ssem = (ss0, ss1)

        def start_gather(g, b):
            pltpu.async_copy(
                table_hbm.at[idx_v.at[pl.ds(g * _SUB, _SUB)]],
                gbuf[b], gsem[b])

        def wait_gather(b):
            pltpu.make_async_copy(
                table_hbm.at[idx_v.at[pl.ds(0, _SUB)]],
                gbuf[b], gsem[b]).wait()

        def scale(b):
            @plsc.parallel_loop(0, _SUB)
            def _s(r):
                for j in range(D_MODEL // _L):
                    sl = pl.ds(j * _L, _L)
                    obuf[b][r, sl] = gbuf[b][r, sl] * SCALE

        def start_scatter(g, b):
            pltpu.async_copy(
                obuf[b], out_hbm.at[pl.ds(base + g * _SUB, _SUB)], ssem[b])

        def wait_scatter(b):
            pltpu.make_async_copy(
                obuf[b], out_hbm.at[pl.ds(0, _SUB)], ssem[b]).wait()

        for b in range(_NBUF):
            start_gather(b, b)

        for g in range(_NBUF):
            b = g
            wait_gather(b)
            scale(b)
            start_scatter(g, b)
            start_gather(g + _NBUF, b)

        @pl.loop(_NBUF, nsub - _NBUF, step=_NBUF)
        def _main(gg):
            for b in range(_NBUF):
                g = gg + b
                wait_gather(b)
                wait_scatter(b)
                scale(b)
                start_scatter(g, b)
                start_gather(g + _NBUF, b)

        for k in range(_NBUF):
            g = nsub - _NBUF + k
            b = g % _NBUF
            wait_gather(b)
            wait_scatter(b)
            scale(b)
            start_scatter(g, b)

        for b in range(_NBUF):
            wait_scatter(b)

    return emb(table, idx)


def kernel(x, table):
    n = x.size
    idx = x.reshape(n).astype(jnp.int32)
    out = _embed_flat(table, idx, n=n)
    return out.reshape(x.shape + (D_MODEL,))
